# baseline (speedup 1.0000x reference)
import jax
import jax.numpy as jnp
from jax import lax
from jax.experimental import pallas as pl
from jax.experimental.pallas import tpu as pltpu
from jax.experimental.pallas import tpu_sc as plsc

_T = 32768
_V = 32
_NW = 32
_CHUNK = _T // _NW
_L = 16
_NG = _CHUNK // _L


def _argmax16(ref, ridx):
    best_v = plsc.load_gather(ref, [ridx, jnp.zeros((_L,), jnp.int32)])
    best_i = jnp.zeros((_L,), jnp.int32)
    for v in range(1, _V):
        x = plsc.load_gather(ref, [ridx, jnp.full((_L,), v, jnp.int32)])
        gt = x > best_v
        best_i = jnp.where(gt, jnp.full((_L,), v, jnp.int32), best_i)
        best_v = jnp.maximum(best_v, x)
    return best_i


_SUB = 128
_NSUB = _CHUNK // _SUB


def _decode_body(em_hbm, out_hbm, blk, halo, ids, outv):
    c = lax.axis_index("c")
    s = lax.axis_index("s")
    wid = s * 2 + c
    start = wid * _CHUNK

    lanes = lax.iota(jnp.int32, _L)

    hs = pl.multiple_of(jnp.maximum(start - 8, 0), 8)
    pltpu.sync_copy(em_hbm.at[pl.ds(hs, 8)], halo)
    p0 = _argmax16(halo, jnp.full((_L,), 7, jnp.int32))
    p0 = jnp.where(jnp.broadcast_to(wid > 0, (_L,)), p0,
                   jnp.full((_L,), -1, jnp.int32))
    plsc.store_scatter(ids, [jnp.zeros((_L,), jnp.int32)], p0,
                       mask=lanes == 0)

    def pass1(j, carry):
        src = pl.multiple_of(start + j * _SUB, 8)
        pltpu.sync_copy(em_hbm.at[pl.ds(src, _SUB)], blk)
        for g in range(_SUB // _L):
            bi = _argmax16(blk, g * _L + lanes)
            plsc.store_scatter(ids, [1 + j * _SUB + g * _L + lanes], bi)
        return carry
    lax.fori_loop(0, _NSUB, pass1, jnp.int32(0))

    def pass2(g, carry):
        prev = ids[pl.ds(g * _L, _L)]
        cur = plsc.load_gather(ids, [g * _L + 1 + lanes])
        keep = (cur != prev) & (cur != 0)
        outv[pl.ds(g * _L, _L)] = jnp.where(keep, cur,
                                            jnp.full((_L,), -1, jnp.int32))
        return carry
    lax.fori_loop(0, _NG, pass2, jnp.int32(0))

    pltpu.sync_copy(outv, out_hbm.at[pl.ds(start, _CHUNK)])


@jax.jit
def kernel(emission):
    mesh = plsc.VectorSubcoreMesh(core_axis_name="c", subcore_axis_name="s")
    f = pl.kernel(
        _decode_body,
        out_type=jax.ShapeDtypeStruct((_T,), jnp.int32),
        mesh=mesh,
        scratch_types=[
            pltpu.VMEM((_SUB, _V), jnp.float32),
            pltpu.VMEM((8, _V), jnp.float32),
            pltpu.VMEM((_CHUNK + _L,), jnp.int32),
            pltpu.VMEM((_CHUNK,), jnp.int32),
        ],
        compiler_params=pltpu.CompilerParams(needs_layout_passes=False),
    )
    return f(emission)


# speedup vs baseline: 1.8122x; 1.8122x over previous
import jax
import jax.numpy as jnp
from jax import lax
from jax.experimental import pallas as pl
from jax.experimental.pallas import tpu as pltpu
from jax.experimental.pallas import tpu_sc as plsc

_T = 32768
_V = 32
_NW = 32
_CHUNK = _T // _NW
_L = 16
_B = 128
_NB = _CHUNK // _B


def _decode_body(em_hbm, out_hbm, buf, ids, outv):
    c = lax.axis_index("c")
    s = lax.axis_index("s")
    wid = s * 2 + c
    start = wid * _CHUNK
    cs = pl.multiple_of(jnp.maximum(start - _B, 0), _B)
    lanes = lax.iota(jnp.int32, _L)

    plsc.store_scatter(ids, [jnp.zeros((_L,), jnp.int32)],
                       jnp.full((_L,), -1, jnp.int32), mask=lanes == 0)

    def pass1(b, carry):
        col = pl.multiple_of(cs + b * _B, _B)
        pltpu.sync_copy(em_hbm.at[:, pl.ds(col, _B)], buf)
        for g in range(_B // _L):
            best_v = buf[0, pl.ds(g * _L, _L)]
            best_i = jnp.zeros((_L,), jnp.int32)
            for v in range(1, _V):
                x = buf[v, pl.ds(g * _L, _L)]
                gt = x > best_v
                best_i = jnp.where(gt, jnp.full((_L,), v, jnp.int32),
                                   best_i)
                best_v = jnp.maximum(best_v, x)
            plsc.store_scatter(ids, [1 + b * _B + g * _L + lanes], best_i)
        return carry
    lax.fori_loop(0, _NB + 1, pass1, jnp.int32(0))

    o = jnp.where(wid > 0, _B + 1, 1).astype(jnp.int32)

    def pass2(g, carry):
        cur = plsc.load_gather(ids, [o + g * _L + lanes])
        prev = plsc.load_gather(ids, [o - 1 + g * _L + lanes])
        keep = (cur != prev) & (cur != 0)
        outv[pl.ds(g * _L, _L)] = jnp.where(keep, cur,
                                            jnp.full((_L,), -1, jnp.int32))
        return carry
    lax.fori_loop(0, _CHUNK // _L, pass2, jnp.int32(0))

    pltpu.sync_copy(outv, out_hbm.at[pl.ds(start, _CHUNK)])


@jax.jit
def kernel(emission):
    mesh = plsc.VectorSubcoreMesh(core_axis_name="c", subcore_axis_name="s")
    f = pl.kernel(
        _decode_body,
        out_type=jax.ShapeDtypeStruct((_T,), jnp.int32),
        mesh=mesh,
        scratch_types=[
            pltpu.VMEM((_V, _B), jnp.float32),
            pltpu.VMEM((1 + (_NB + 1) * _B + _L,), jnp.int32),
            pltpu.VMEM((_CHUNK,), jnp.int32),
        ],
        compiler_params=pltpu.CompilerParams(needs_layout_passes=False),
    )
    return f(emission.T)


# speedup vs baseline: 2.1803x; 1.2032x over previous
import jax
import jax.numpy as jnp
from jax import lax
from jax.experimental import pallas as pl
from jax.experimental.pallas import tpu as pltpu
from jax.experimental.pallas import tpu_sc as plsc

_T = 32768
_V = 32
_NW = 32
_CHUNK = _T // _NW
_L = 16
_B = 128
_NB = _CHUNK // _B


def _decode_body(em_hbm, out_hbm, buf, ids, outv):
    c = lax.axis_index("c")
    s = lax.axis_index("s")
    wid = s * 2 + c
    start = wid * _CHUNK
    cs = pl.multiple_of(jnp.maximum(start - _B, 0), _B)
    lanes = lax.iota(jnp.int32, _L)

    plsc.store_scatter(ids, [jnp.zeros((_L,), jnp.int32)],
                       jnp.full((_L,), -1, jnp.int32), mask=lanes == 0)

    pltpu.sync_copy(em_hbm.at[:, pl.ds(cs, (_NB + 1) * _B)], buf)

    def pass1(b, carry):
        for g in range(_B // _L):
            col = b * _B + g * _L
            best_v = buf[0, pl.ds(col, _L)]
            best_i = jnp.zeros((_L,), jnp.int32)
            for v in range(1, _V):
                x = buf[v, pl.ds(col, _L)]
                gt = x > best_v
                best_i = jnp.where(gt, jnp.full((_L,), v, jnp.int32),
                                   best_i)
                best_v = jnp.maximum(best_v, x)
            plsc.store_scatter(ids, [1 + col + lanes], best_i)
        return carry
    lax.fori_loop(0, _NB + 1, pass1, jnp.int32(0))

    o = jnp.where(wid > 0, _B + 1, 1).astype(jnp.int32)

    def pass2(g, carry):
        cur = plsc.load_gather(ids, [o + g * _L + lanes])
        prev = plsc.load_gather(ids, [o - 1 + g * _L + lanes])
        keep = (cur != prev) & (cur != 0)
        outv[pl.ds(g * _L, _L)] = jnp.where(keep, cur,
                                            jnp.full((_L,), -1, jnp.int32))
        return carry
    lax.fori_loop(0, _CHUNK // _L, pass2, jnp.int32(0))

    pltpu.sync_copy(outv, out_hbm.at[pl.ds(start, _CHUNK)])


@jax.jit
def kernel(emission):
    mesh = plsc.VectorSubcoreMesh(core_axis_name="c", subcore_axis_name="s")
    f = pl.kernel(
        _decode_body,
        out_type=jax.ShapeDtypeStruct((_T,), jnp.int32),
        mesh=mesh,
        scratch_types=[
            pltpu.VMEM((_V, (_NB + 1) * _B), jnp.float32),
            pltpu.VMEM((1 + (_NB + 1) * _B + _L,), jnp.int32),
            pltpu.VMEM((_CHUNK,), jnp.int32),
        ],
        compiler_params=pltpu.CompilerParams(needs_layout_passes=False),
    )
    return f(emission.T)


# speedup vs baseline: 2.2313x; 1.0234x over previous
import jax
import jax.numpy as jnp
from jax import lax
from jax.experimental import pallas as pl
from jax.experimental.pallas import tpu as pltpu
from jax.experimental.pallas import tpu_sc as plsc

_T = 32768
_V = 32
_NW = 32
_CHUNK = _T // _NW
_L = 16
_B = 128
_WA = 512
_WB = 640


def _argmax_pass(buf, ids, ngroups, base, carry_unused):
    lanes = lax.iota(jnp.int32, _L)

    def body(g, carry):
        col = g * _L
        best_v = buf[0, pl.ds(col, _L)]
        best_i = jnp.zeros((_L,), jnp.int32)
        for v in range(1, _V):
            x = buf[v, pl.ds(col, _L)]
            gt = x > best_v
            best_i = jnp.where(gt, jnp.full((_L,), v, jnp.int32), best_i)
            best_v = jnp.maximum(best_v, x)
        plsc.store_scatter(ids, [base + col + lanes], best_i)
        return carry
    return lax.fori_loop(0, ngroups, body, carry_unused)


def _decode_body(em_hbm, out_hbm, bufa, bufb, ids, outv, sema, semb):
    c = lax.axis_index("c")
    s = lax.axis_index("s")
    wid = s * 2 + c
    start = wid * _CHUNK
    cs = pl.multiple_of(jnp.maximum(start - _B, 0), _B)
    lanes = lax.iota(jnp.int32, _L)

    cpa = pltpu.async_copy(em_hbm.at[:, pl.ds(cs, _WA)], bufa, sema)
    cpb = pltpu.async_copy(
        em_hbm.at[:, pl.ds(pl.multiple_of(cs + _WA, _B), _WB)], bufb, semb)

    plsc.store_scatter(ids, [jnp.zeros((_L,), jnp.int32)],
                       jnp.full((_L,), -1, jnp.int32), mask=lanes == 0)

    cpa.wait()
    _argmax_pass(bufa, ids, _WA // _L, 1, jnp.int32(0))
    cpb.wait()
    _argmax_pass(bufb, ids, _WB // _L, 1 + _WA, jnp.int32(0))

    o = jnp.where(wid > 0, _B + 1, 1).astype(jnp.int32)

    def pass2(g, carry):
        cur = plsc.load_gather(ids, [o + g * _L + lanes])
        prev = plsc.load_gather(ids, [o - 1 + g * _L + lanes])
        keep = (cur != prev) & (cur != 0)
        outv[pl.ds(g * _L, _L)] = jnp.where(keep, cur,
                                            jnp.full((_L,), -1, jnp.int32))
        return carry
    lax.fori_loop(0, _CHUNK // _L, pass2, jnp.int32(0))

    pltpu.sync_copy(outv, out_hbm.at[pl.ds(start, _CHUNK)])


@jax.jit
def kernel(emission):
    mesh = plsc.VectorSubcoreMesh(core_axis_name="c", subcore_axis_name="s")
    f = pl.kernel(
        _decode_body,
        out_type=jax.ShapeDtypeStruct((_T,), jnp.int32),
        mesh=mesh,
        scratch_types=[
            pltpu.VMEM((_V, _WA), jnp.float32),
            pltpu.VMEM((_V, _WB), jnp.float32),
            pltpu.VMEM((1 + _B + _CHUNK + _L,), jnp.int32),
            pltpu.VMEM((_CHUNK,), jnp.int32),
            pltpu.SemaphoreType.DMA,
            pltpu.SemaphoreType.DMA,
        ],
        compiler_params=pltpu.CompilerParams(needs_layout_passes=False),
    )
    return f(emission.T)
